# baseline (speedup 1.0000x reference)
import jax
import jax.numpy as jnp
from jax import lax
from jax.experimental import pallas as pl
from jax.experimental.pallas import tpu as pltpu
from jax.experimental.pallas import tpu_sc as plsc

D = 4096
BATCH = 128
NC, NS = 2, 16
NW = NC * NS
ROWS_W = D // NW
LANES = 16

_INV_2PI = 0.15915493667125702
_C1 = 6.28125
_C2 = 1.9353071795864769e-3
_PI = 3.14159265358979
_PI_2 = 1.5707963267948966


def _sincos16(a):
    t = a * _INV_2PI
    t = t + jnp.where(t >= 0.0, 0.5, -0.5)
    kf = t.astype(jnp.int32).astype(jnp.float32)
    r = a - kf * _C1
    r = r - kf * _C2
    flip = jnp.abs(r) > _PI_2
    half_turn = jnp.where(r >= 0.0, _PI, -_PI)
    rf = jnp.where(flip, half_turn - r, r)
    r2 = rf * rf
    s = rf * (1.0 + r2 * (-1.0 / 6.0 + r2 * (1.0 / 120.0 + r2 * (
        -1.0 / 5040.0 + r2 * (1.0 / 362880.0 + r2 * (-1.0 / 39916800.0))))))
    c = 1.0 + r2 * (-0.5 + r2 * (1.0 / 24.0 + r2 * (-1.0 / 720.0 + r2 * (
        1.0 / 40320.0 + r2 * (-1.0 / 3628800.0 + r2 * (1.0 / 479001600.0))))))
    c = jnp.where(flip, -c, c)
    return s, c


def _crz_body(x_hbm, ang_hbm, re_hbm, im_hbm, x_v, re_v, im_v, ang_v):
    w = lax.axis_index("s") * NC + lax.axis_index("c")
    base = w * ROWS_W
    pltpu.sync_copy(ang_hbm, ang_v)
    pltpu.sync_copy(x_hbm.at[pl.ds(base, ROWS_W)], x_v)

    @pl.when(w < 16)
    def _identity_rows():
        zv = jnp.zeros((LANES,), jnp.float32)

        def zbody(r, carry):
            for u in range(BATCH // LANES):
                im_v[r, pl.ds(u * LANES, LANES)] = zv
            return carry

        lax.fori_loop(0, ROWS_W, zbody, 0)
        pltpu.sync_copy(x_v, re_hbm.at[pl.ds(base, ROWS_W)])
        pltpu.sync_copy(im_v, im_hbm.at[pl.ds(base, ROWS_W)])

    @pl.when(w >= 16)
    def _gate_rows():
        sinv, cosv = _sincos16(ang_v[...] * 0.5)
        m_mid = lax.convert_element_type(w < 24, jnp.float32)
        svec = (1.0 - 2.0 * m_mid) * sinv

        def body(r, carry):
            for u in range(BATCH // LANES):
                sl = pl.ds(u * LANES, LANES)
                v = x_v[r, sl]
                re_v[r, sl] = v * cosv
                im_v[r, sl] = v * svec
            return carry

        lax.fori_loop(0, ROWS_W, body, 0)
        pltpu.sync_copy(re_v, re_hbm.at[pl.ds(base, ROWS_W)])
        pltpu.sync_copy(im_v, im_hbm.at[pl.ds(base, ROWS_W)])


def _build_crz_sc():
    mesh = plsc.VectorSubcoreMesh(
        core_axis_name="c", subcore_axis_name="s",
        num_cores=NC, num_subcores=NS)
    return pl.kernel(
        _crz_body,
        compiler_params=pltpu.CompilerParams(use_tc_tiling_on_sc=True),
        out_type=(
            jax.ShapeDtypeStruct((D, BATCH), jnp.float32),
            jax.ShapeDtypeStruct((D, BATCH), jnp.float32),
        ),
        mesh=mesh,
        scratch_types=[
            pltpu.VMEM((ROWS_W, BATCH), jnp.float32),
            pltpu.VMEM((ROWS_W, BATCH), jnp.float32),
            pltpu.VMEM((ROWS_W, BATCH), jnp.float32),
            pltpu.VMEM((LANES,), jnp.float32),
        ],
    )


def kernel(x, angle):
    ang16 = jnp.broadcast_to(angle.astype(jnp.float32), (LANES,))
    re, im = _build_crz_sc()(x, ang16)
    return lax.complex(re, im)


# speedup vs baseline: 1.0027x; 1.0027x over previous
import jax
import jax.numpy as jnp
from jax import lax
from jax.experimental import pallas as pl
from jax.experimental.pallas import tpu as pltpu
from jax.experimental.pallas import tpu_sc as plsc

D = 4096
BATCH = 128
NC, NS = 2, 16
NW = NC * NS
ROWS_W = D // NW
HALF = ROWS_W // 2
LANES = 16

_INV_2PI = 0.15915493667125702
_C1 = 6.28125
_C2 = 1.9353071795864769e-3
_PI = 3.14159265358979
_PI_2 = 1.5707963267948966


def _sincos16(a):
    t = a * _INV_2PI
    t = t + jnp.where(t >= 0.0, 0.5, -0.5)
    kf = t.astype(jnp.int32).astype(jnp.float32)
    r = a - kf * _C1
    r = r - kf * _C2
    flip = jnp.abs(r) > _PI_2
    half_turn = jnp.where(r >= 0.0, _PI, -_PI)
    rf = jnp.where(flip, half_turn - r, r)
    r2 = rf * rf
    s = rf * (1.0 + r2 * (-1.0 / 6.0 + r2 * (1.0 / 120.0 + r2 * (
        -1.0 / 5040.0 + r2 * (1.0 / 362880.0 + r2 * (-1.0 / 39916800.0))))))
    c = 1.0 + r2 * (-0.5 + r2 * (1.0 / 24.0 + r2 * (-1.0 / 720.0 + r2 * (
        1.0 / 40320.0 + r2 * (-1.0 / 3628800.0 + r2 * (1.0 / 479001600.0))))))
    c = jnp.where(flip, -c, c)
    return s, c


def _crz_body(x_hbm, ang_hbm, re_hbm, im_hbm,
              x_v, re_v, im_v, ang_v, s_ang, s_in0, s_in1, s_out):
    w = lax.axis_index("s") * NC + lax.axis_index("c")
    base = w * ROWS_W

    ca = pltpu.async_copy(ang_hbm, ang_v, s_ang)
    c0 = pltpu.async_copy(x_hbm.at[pl.ds(base, HALF)],
                          x_v.at[pl.ds(0, HALF)], s_in0)
    c1 = pltpu.async_copy(x_hbm.at[pl.ds(base + HALF, HALF)],
                          x_v.at[pl.ds(HALF, HALF)], s_in1)

    @pl.when(w < 16)
    def _identity_rows():
        zv = jnp.zeros((LANES,), jnp.float32)

        @plsc.parallel_loop(0, ROWS_W, unroll=2)
        def _zero(r):
            for u in range(BATCH // LANES):
                im_v[r, pl.ds(u * LANES, LANES)] = zv

        ca.wait()
        c0.wait()
        c1.wait()
        o_re = pltpu.async_copy(x_v, re_hbm.at[pl.ds(base, ROWS_W)], s_out)
        o_im = pltpu.async_copy(im_v, im_hbm.at[pl.ds(base, ROWS_W)], s_out)
        o_re.wait()
        o_im.wait()

    @pl.when(w >= 16)
    def _gate_rows():
        ca.wait()
        sinv, cosv = _sincos16(ang_v[...] * 0.5)
        m_mid = lax.convert_element_type(w < 24, jnp.float32)
        svec = (1.0 - 2.0 * m_mid) * sinv

        def compute(lo):
            @plsc.parallel_loop(lo, lo + HALF, unroll=2)
            def _scale(r):
                for u in range(BATCH // LANES):
                    sl = pl.ds(u * LANES, LANES)
                    v = x_v[r, sl]
                    re_v[r, sl] = v * cosv
                    im_v[r, sl] = v * svec

        c0.wait()
        compute(0)
        o0r = pltpu.async_copy(re_v.at[pl.ds(0, HALF)],
                               re_hbm.at[pl.ds(base, HALF)], s_out)
        o0i = pltpu.async_copy(im_v.at[pl.ds(0, HALF)],
                               im_hbm.at[pl.ds(base, HALF)], s_out)
        c1.wait()
        compute(HALF)
        o1r = pltpu.async_copy(re_v.at[pl.ds(HALF, HALF)],
                               re_hbm.at[pl.ds(base + HALF, HALF)], s_out)
        o1i = pltpu.async_copy(im_v.at[pl.ds(HALF, HALF)],
                               im_hbm.at[pl.ds(base + HALF, HALF)], s_out)
        o0r.wait()
        o0i.wait()
        o1r.wait()
        o1i.wait()


def _build_crz_sc():
    mesh = plsc.VectorSubcoreMesh(
        core_axis_name="c", subcore_axis_name="s",
        num_cores=NC, num_subcores=NS)
    return pl.kernel(
        _crz_body,
        out_type=(
            jax.ShapeDtypeStruct((D, BATCH), jnp.float32),
            jax.ShapeDtypeStruct((D, BATCH), jnp.float32),
        ),
        mesh=mesh,
        scratch_types=[
            pltpu.VMEM((ROWS_W, BATCH), jnp.float32),
            pltpu.VMEM((ROWS_W, BATCH), jnp.float32),
            pltpu.VMEM((ROWS_W, BATCH), jnp.float32),
            pltpu.VMEM((LANES,), jnp.float32),
            pltpu.SemaphoreType.DMA,
            pltpu.SemaphoreType.DMA,
            pltpu.SemaphoreType.DMA,
            pltpu.SemaphoreType.DMA,
        ],
    )


def kernel(x, angle):
    ang16 = jnp.broadcast_to(angle.astype(jnp.float32), (LANES,))
    re, im = _build_crz_sc()(x, ang16)
    return lax.complex(re, im)


# speedup vs baseline: 1.0070x; 1.0042x over previous
import jax
import jax.numpy as jnp
from jax import lax
from jax.experimental import pallas as pl
from jax.experimental.pallas import tpu as pltpu
from jax.experimental.pallas import tpu_sc as plsc

D = 4096
BATCH = 128
NC, NS = 2, 16
NW = NC * NS
ROWS_W = D // NW
HALF = ROWS_W // 2
LANES = 16

_INV_2PI = 0.15915493667125702
_C1 = 6.28125
_C2 = 1.9353071795864769e-3
_PI = 3.14159265358979
_PI_2 = 1.5707963267948966


def _sincos16(a):
    t = a * _INV_2PI
    t = t + jnp.where(t >= 0.0, 0.5, -0.5)
    kf = t.astype(jnp.int32).astype(jnp.float32)
    r = a - kf * _C1
    r = r - kf * _C2
    flip = jnp.abs(r) > _PI_2
    half_turn = jnp.where(r >= 0.0, _PI, -_PI)
    rf = jnp.where(flip, half_turn - r, r)
    r2 = rf * rf
    s = rf * (1.0 + r2 * (-1.0 / 6.0 + r2 * (1.0 / 120.0 + r2 * (
        -1.0 / 5040.0 + r2 * (1.0 / 362880.0 + r2 * (-1.0 / 39916800.0))))))
    c = 1.0 + r2 * (-0.5 + r2 * (1.0 / 24.0 + r2 * (-1.0 / 720.0 + r2 * (
        1.0 / 40320.0 + r2 * (-1.0 / 3628800.0 + r2 * (1.0 / 479001600.0))))))
    c = jnp.where(flip, -c, c)
    return s, c


def _crz_body(x_hbm, ang_hbm, re_hbm, im_hbm,
              x_v, re_v, im_v, ang_v, s_ang, s_in, s_out):
    w = lax.axis_index("s") * NC + lax.axis_index("c")
    base = w * ROWS_W

    ca = pltpu.async_copy(ang_hbm, ang_v, s_ang)
    cx = pltpu.async_copy(x_hbm.at[pl.ds(base, ROWS_W)], x_v, s_in)

    ca.wait()
    sinv, cosv = _sincos16(ang_v[...] * 0.5)
    m_low = lax.convert_element_type(w < 16, jnp.float32)
    m_mid = lax.convert_element_type(w < 24, jnp.float32)
    cvec = m_low + (1.0 - m_low) * cosv
    svec = ((1.0 - m_low) * (1.0 - 2.0 * m_mid)) * sinv

    cx.wait()

    def body(r, carry):
        for u in range(BATCH // LANES):
            sl = pl.ds(u * LANES, LANES)
            v = x_v[r, sl]
            re_v[r, sl] = v * cvec
            im_v[r, sl] = v * svec
        return carry

    lax.fori_loop(0, ROWS_W, body, 0)

    o_re = pltpu.async_copy(re_v, re_hbm.at[pl.ds(base, ROWS_W)], s_out)
    o_im = pltpu.async_copy(im_v, im_hbm.at[pl.ds(base, ROWS_W)], s_out)
    o_re.wait()
    o_im.wait()


def _build_crz_sc():
    mesh = plsc.VectorSubcoreMesh(
        core_axis_name="c", subcore_axis_name="s",
        num_cores=NC, num_subcores=NS)
    return pl.kernel(
        _crz_body,
        out_type=(
            jax.ShapeDtypeStruct((D, BATCH), jnp.float32),
            jax.ShapeDtypeStruct((D, BATCH), jnp.float32),
        ),
        mesh=mesh,
        scratch_types=[
            pltpu.VMEM((ROWS_W, BATCH), jnp.float32),
            pltpu.VMEM((ROWS_W, BATCH), jnp.float32),
            pltpu.VMEM((ROWS_W, BATCH), jnp.float32),
            pltpu.VMEM((LANES,), jnp.float32),
            pltpu.SemaphoreType.DMA,
            pltpu.SemaphoreType.DMA,
            pltpu.SemaphoreType.DMA,
        ],
    )


def kernel(x, angle):
    ang16 = jnp.broadcast_to(angle.astype(jnp.float32), (LANES,))
    re, im = _build_crz_sc()(x, ang16)
    return lax.complex(re, im)


# speedup vs baseline: 1.0102x; 1.0032x over previous
import jax
import jax.numpy as jnp
from jax import lax
from jax.experimental import pallas as pl
from jax.experimental.pallas import tpu as pltpu
from jax.experimental.pallas import tpu_sc as plsc

D = 4096
BATCH = 128
NC, NS = 2, 16
NW = NC * NS
ROWS_W = D // NW
HALF = ROWS_W // 2
LANES = 16

_INV_2PI = 0.15915493667125702
_C1 = 6.28125
_C2 = 1.9353071795864769e-3
_PI = 3.14159265358979
_PI_2 = 1.5707963267948966


def _sincos16(a):
    t = a * _INV_2PI
    t = t + jnp.where(t >= 0.0, 0.5, -0.5)
    kf = t.astype(jnp.int32).astype(jnp.float32)
    r = a - kf * _C1
    r = r - kf * _C2
    flip = jnp.abs(r) > _PI_2
    half_turn = jnp.where(r >= 0.0, _PI, -_PI)
    rf = jnp.where(flip, half_turn - r, r)
    r2 = rf * rf
    s = rf * (1.0 + r2 * (-1.0 / 6.0 + r2 * (1.0 / 120.0 + r2 * (
        -1.0 / 5040.0 + r2 * (1.0 / 362880.0 + r2 * (-1.0 / 39916800.0))))))
    c = 1.0 + r2 * (-0.5 + r2 * (1.0 / 24.0 + r2 * (-1.0 / 720.0 + r2 * (
        1.0 / 40320.0 + r2 * (-1.0 / 3628800.0 + r2 * (1.0 / 479001600.0))))))
    c = jnp.where(flip, -c, c)
    return s, c


def _crz_body(x_hbm, ang_hbm, re_hbm, im_hbm,
              x_v, re_v, im_v, ang_v, s_ang, s_in, s_out):
    w = lax.axis_index("s") * NC + lax.axis_index("c")
    base = w * ROWS_W

    ca = pltpu.async_copy(ang_hbm, ang_v, s_ang)
    c0 = pltpu.async_copy(x_hbm.at[pl.ds(base, HALF)],
                          x_v.at[pl.ds(0, HALF)], s_in)
    c1 = pltpu.async_copy(x_hbm.at[pl.ds(base + HALF, HALF)],
                          x_v.at[pl.ds(HALF, HALF)], s_in)

    ca.wait()
    sinv, cosv = _sincos16(ang_v[...] * 0.5)
    m_low = lax.convert_element_type(w < 16, jnp.float32)
    m_mid = lax.convert_element_type(w < 24, jnp.float32)
    cvec = m_low + (1.0 - m_low) * cosv
    svec = ((1.0 - m_low) * (1.0 - 2.0 * m_mid)) * sinv

    def body(r, carry):
        for u in range(BATCH // LANES):
            sl = pl.ds(u * LANES, LANES)
            v = x_v[r, sl]
            re_v[r, sl] = v * cvec
            im_v[r, sl] = v * svec
        return carry

    c0.wait()
    lax.fori_loop(0, HALF, body, 0)
    o0r = pltpu.async_copy(re_v.at[pl.ds(0, HALF)],
                           re_hbm.at[pl.ds(base, HALF)], s_out)
    o0i = pltpu.async_copy(im_v.at[pl.ds(0, HALF)],
                           im_hbm.at[pl.ds(base, HALF)], s_out)
    c1.wait()
    lax.fori_loop(HALF, ROWS_W, body, 0)
    o1r = pltpu.async_copy(re_v.at[pl.ds(HALF, HALF)],
                           re_hbm.at[pl.ds(base + HALF, HALF)], s_out)
    o1i = pltpu.async_copy(im_v.at[pl.ds(HALF, HALF)],
                           im_hbm.at[pl.ds(base + HALF, HALF)], s_out)
    o0r.wait()
    o0i.wait()
    o1r.wait()
    o1i.wait()


def _build_crz_sc():
    mesh = plsc.VectorSubcoreMesh(
        core_axis_name="c", subcore_axis_name="s",
        num_cores=NC, num_subcores=NS)
    return pl.kernel(
        _crz_body,
        out_type=(
            jax.ShapeDtypeStruct((D, BATCH), jnp.float32),
            jax.ShapeDtypeStruct((D, BATCH), jnp.float32),
        ),
        mesh=mesh,
        scratch_types=[
            pltpu.VMEM((ROWS_W, BATCH), jnp.float32),
            pltpu.VMEM((ROWS_W, BATCH), jnp.float32),
            pltpu.VMEM((ROWS_W, BATCH), jnp.float32),
            pltpu.VMEM((LANES,), jnp.float32),
            pltpu.SemaphoreType.DMA,
            pltpu.SemaphoreType.DMA,
            pltpu.SemaphoreType.DMA,
        ],
    )


def kernel(x, angle):
    ang16 = jnp.broadcast_to(angle.astype(jnp.float32), (LANES,))
    re, im = _build_crz_sc()(x, ang16)
    return lax.complex(re, im)
